# Initial kernel scaffold; baseline (speedup 1.0000x reference)
import jax
import jax.numpy as jnp
from jax.experimental import pallas as pl


def kernel(inputs):
    raise NotImplementedError("write your pallas kernel here")


# speedup vs baseline: 10.5854x; 10.5854x over previous
import functools

import jax
import jax.numpy as jnp
from jax import lax
from jax.experimental import pallas as pl
from jax.experimental.pallas import tpu as pltpu
from jax.experimental.pallas import tpu_sc as plsc

_W = 32
_BLK_PATCH = 16
_MAX_PASSES = 64


def _roll(x, shift, axis):
    n = x.shape[axis]
    shift = shift % n
    if shift == 0:
        return x
    a = lax.slice_in_dim(x, n - shift, n, axis=axis)
    b = lax.slice_in_dim(x, 0, n - shift, axis=axis)
    return lax.concatenate([a, b], dimension=axis)


def _seg_min_scan(v, c, axis, sign):
    d = 1
    while d < _W:
        vs = _roll(v, sign * d, axis)
        cs = _roll(c, sign * d, axis)
        v = jnp.where(c != 0, jnp.minimum(v, vs), v)
        c = c & cs
        d *= 2
    return v


def _ccl_kernel(x_ref, out_ref):
    x = x_ref[...]
    mask = x != 0.0
    rows, cols = x.shape
    l = lax.broadcasted_iota(jnp.int32, (rows, cols), 1)
    r = lax.broadcasted_iota(jnp.int32, (rows, cols), 0)
    col = l & (_W - 1)
    lbl0 = jnp.where(mask, r * _W + col + 1, 0)

    mi = mask.astype(jnp.int32)
    mh_f = mi & _roll(mi, 1, 1) & (col != 0).astype(jnp.int32)
    mh_b = mi & _roll(mi, -1, 1) & (col != _W - 1).astype(jnp.int32)
    mv_f = mi & _roll(mi, 1, 0) & (r != 0).astype(jnp.int32)
    mv_b = mi & _roll(mi, -1, 0) & (r != rows - 1).astype(jnp.int32)

    def one_pass(v):
        v = _seg_min_scan(v, mh_f, 1, 1)
        v = _seg_min_scan(v, mh_b, 1, -1)
        v = _seg_min_scan(v, mv_f, 0, 1)
        v = _seg_min_scan(v, mv_b, 0, -1)
        return v

    def cond(st):
        i, changed, _ = st
        return changed & (i < _MAX_PASSES)

    def body(st):
        i, _, v = st
        nv = one_pass(v)
        return (i + 1, jnp.any(nv != v), nv)

    _, _, lbl = lax.while_loop(cond, body, (jnp.int32(0), True, lbl0))
    out_ref[...] = lbl


def _run_ccl(xt, interpret=False):
    n = xt.shape[1] // (_BLK_PATCH * _W)
    return pl.pallas_call(
        _ccl_kernel,
        grid=(n,),
        in_specs=[pl.BlockSpec((_W, _BLK_PATCH * _W), lambda i: (0, i))],
        out_specs=pl.BlockSpec((_W, _BLK_PATCH * _W), lambda i: (0, i)),
        out_shape=jax.ShapeDtypeStruct(xt.shape, jnp.int32),
        interpret=interpret,
    )(xt)


def _make_count(n_groups):
    mesh = plsc.VectorSubcoreMesh(core_axis_name="c", subcore_axis_name="s")
    n_tiles = 32
    rounds = (n_groups + n_tiles - 1) // n_tiles

    @functools.partial(
        pl.kernel,
        out_type=jax.ShapeDtypeStruct((n_groups, 16), jnp.float32),
        mesh=mesh,
        compiler_params=pltpu.CompilerParams(needs_layout_passes=False),
        scratch_types=[
            pltpu.VMEM((_W, _W * _W), jnp.int32),
            pltpu.VMEM((1040,), jnp.int32),
            pltpu.VMEM((16,), jnp.float32),
        ],
    )
    def count_kernel(lbl_hbm, out_hbm, lblv, cnt, outv):
        wid = lax.axis_index("s") * 2 + lax.axis_index("c")
        zero16 = jnp.zeros((16,), jnp.int32)
        ones16 = jnp.ones((16,), jnp.int32)

        for gi in range(rounds):
            g = wid + n_tiles * gi

            @pl.when(g < n_groups)
            def _():
                pltpu.sync_copy(lbl_hbm.at[:, pl.ds(g * _W * _W, _W * _W)],
                                lblv)

                def patch_body(pp, gsum):
                    def zbody(k, carry):
                        cnt[pl.ds(k * 16, 16)] = zero16
                        return carry
                    lax.fori_loop(0, 65, zbody, 0)

                    def sbody(j, carry):
                        rr = j >> 1
                        hh = j & 1
                        vec = lblv[rr, pl.ds(pp * _W + hh * 16, 16)]
                        plsc.addupdate_scatter(cnt, [vec], ones16)
                        return carry
                    lax.fori_loop(0, 2 * _W, sbody, 0)

                    def mbody(k, mx):
                        return jnp.maximum(mx, cnt[pl.ds(k * 16, 16)])
                    mxv = lax.fori_loop(0, 65, mbody, zero16)
                    return gsum + jnp.max(mxv)

                gsum = lax.fori_loop(0, _W, patch_body, jnp.int32(0))
                res = (gsum >> 5).astype(jnp.float32)
                outv[...] = jnp.zeros((16,), jnp.float32) + res
                pltpu.sync_copy(outv, out_hbm.at[g])

    return count_kernel


def kernel(inputs):
    n_cd, n_box, batch, n_patch, h, w = inputs.shape
    x = inputs.reshape(-1, h, w)
    xt = x.transpose(1, 0, 2).reshape(h, -1)
    lbl = _run_ccl(xt)
    n_groups = x.shape[0] // n_patch
    out = _make_count(n_groups)(lbl)
    return out[:, 0].reshape(n_cd, n_box, batch)
